# baseline (speedup 1.0000x reference)
import functools

import jax
import jax.numpy as jnp
from jax import lax
from jax.experimental import pallas as pl
from jax.experimental.pallas import tpu as pltpu
from jax.experimental.pallas import tpu_sc as plsc

_T = 8192
_D = 1024
_B = 4
_NW = 32
_CH = 32
_CE = _CH * _D
_TPW = _T // _NW
_NCH = _TPW // _CH
_U = 16


def _sc_body(x_hbm, pe_hbm, out_hbm, xbuf, pebuf):
    c = lax.axis_index("c")
    s = lax.axis_index("s")
    wid = s * 2 + c
    pe_base = wid * _TPW * _D

    def chunk(i, _):
        pe_off = pe_base + i * _CE
        pltpu.sync_copy(pe_hbm.at[pl.ds(pe_off, _CE)], pebuf)
        for b in range(_B):
            x_off = b * _T * _D + pe_off
            pltpu.sync_copy(x_hbm.at[pl.ds(x_off, _CE)], xbuf)

            def vloop(k, _):
                o = pl.multiple_of(k * (16 * _U), 16 * _U)
                for u in range(_U):
                    pv = pebuf[pl.ds(o + u * 16, 16)]
                    plsc.addupdate(xbuf.at[pl.ds(o + u * 16, 16)], pv)
                return _

            lax.fori_loop(0, _CE // (16 * _U), vloop, None)
            pltpu.sync_copy(xbuf, out_hbm.at[pl.ds(x_off, _CE)])
        return _

    lax.fori_loop(0, _NCH, chunk, None)


def _sc_add(xf, pe):
    n = xf.shape[0]
    return pl.kernel(
        _sc_body,
        out_type=jax.ShapeDtypeStruct((n,), jnp.float32),
        mesh=plsc.VectorSubcoreMesh(core_axis_name="c", subcore_axis_name="s"),
        scratch_types=[
            pltpu.VMEM((_CE,), jnp.float32),
            pltpu.VMEM((_CE,), jnp.float32),
        ],
    )(xf, pe)


def kernel(x, pe):
    B, T, D = x.shape
    out = _sc_add(x.reshape(-1), pe.reshape(-1))
    return out.reshape(B, T, D)


# speedup vs baseline: 1.2021x; 1.2021x over previous
import jax
import jax.numpy as jnp
from jax import lax
from jax.experimental import pallas as pl
from jax.experimental.pallas import tpu as pltpu
from jax.experimental.pallas import tpu_sc as plsc

_T = 8192
_D = 1024
_B = 4
_NW = 32
_CH = 16
_CE = _CH * _D
_TPW = _T // _NW
_NCH = _TPW // _CH
_STEPS = _NCH * _B
_U = 16


def _sc_body(x_hbm, pe_hbm, out_hbm,
             xb0, xb1, xb2, pb0, pb1,
             sx0, sx1, sx2, sp0, sp1, so0, so1, so2):
    xbufs = (xb0, xb1, xb2)
    pbufs = (pb0, pb1)
    sxs = (sx0, sx1, sx2)
    sps = (sp0, sp1)
    sos = (so0, so1, so2)

    c = lax.axis_index("c")
    s = lax.axis_index("s")
    wid = s * 2 + c
    pe_base = wid * _TPW * _D

    def pe_off(i):
        return pe_base + i * _CE

    def x_off(k):
        i, b = divmod(k, _B)
        return b * _T * _D + pe_base + i * _CE

    x_desc = [None] * _STEPS
    o_desc = [None] * _STEPS
    p_desc = [None] * _NCH

    def load_x(k):
        x_desc[k] = pltpu.async_copy(
            x_hbm.at[pl.ds(x_off(k), _CE)], xbufs[k % 3], sxs[k % 3])

    def load_pe(i):
        p_desc[i] = pltpu.async_copy(
            pe_hbm.at[pl.ds(pe_off(i), _CE)], pbufs[i % 2], sps[i % 2])

    load_pe(0)
    load_pe(1)
    load_x(0)
    load_x(1)

    for k in range(_STEPS):
        i, b = divmod(k, _B)
        x_desc[k].wait()
        if b == 0:
            p_desc[i].wait()
        xb = xbufs[k % 3]
        pb = pbufs[i % 2]

        def vloop(j, carry, xb=xb, pb=pb):
            o = pl.multiple_of(j * (16 * _U), 16 * _U)
            for u in range(_U):
                pv = pb[pl.ds(o + u * 16, 16)]
                plsc.addupdate(xb.at[pl.ds(o + u * 16, 16)], pv)
            return carry

        lax.fori_loop(0, _CE // (16 * _U), vloop, None)
        o_desc[k] = pltpu.async_copy(
            xb, out_hbm.at[pl.ds(x_off(k), _CE)], sos[k % 3])
        if k + 2 < _STEPS:
            if k - 1 >= 0:
                o_desc[k - 1].wait()
            load_x(k + 2)
        if b == _B - 1 and i + 2 < _NCH:
            load_pe(i + 2)

    o_desc[_STEPS - 3].wait()
    o_desc[_STEPS - 2].wait()
    o_desc[_STEPS - 1].wait()


def _sc_add(xf, pe):
    n = xf.shape[0]
    return pl.kernel(
        _sc_body,
        out_type=jax.ShapeDtypeStruct((n,), jnp.float32),
        mesh=plsc.VectorSubcoreMesh(core_axis_name="c", subcore_axis_name="s"),
        scratch_types=[
            pltpu.VMEM((_CE,), jnp.float32),
            pltpu.VMEM((_CE,), jnp.float32),
            pltpu.VMEM((_CE,), jnp.float32),
            pltpu.VMEM((_CE,), jnp.float32),
            pltpu.VMEM((_CE,), jnp.float32),
            pltpu.SemaphoreType.DMA,
            pltpu.SemaphoreType.DMA,
            pltpu.SemaphoreType.DMA,
            pltpu.SemaphoreType.DMA,
            pltpu.SemaphoreType.DMA,
            pltpu.SemaphoreType.DMA,
            pltpu.SemaphoreType.DMA,
            pltpu.SemaphoreType.DMA,
        ],
    )(xf, pe)


def kernel(x, pe):
    B, T, D = x.shape
    out = _sc_add(x.reshape(-1), pe.reshape(-1))
    return out.reshape(B, T, D)
